# baseline (speedup 1.0000x reference)
import jax
import jax.numpy as jnp
from jax.experimental import pallas as pl
from jax.experimental.pallas import tpu as pltpu

_BB = 2048


def _mlp_kernel(x_ref, w0a_ref, w0b_ref, b0_ref, w1_ref, b1_ref, out_ref):
    h = jnp.dot(x_ref[0], w0a_ref[...], preferred_element_type=jnp.float32)
    h += jnp.dot(x_ref[1], w0b_ref[...], preferred_element_type=jnp.float32)
    h = jnp.maximum(h + b0_ref[...], 0.0)
    out_ref[...] = jnp.sum(h * w1_ref[...], axis=1, keepdims=True) + b1_ref[...]


def kernel(inputs, W0, b0, W1, b1):
    _, batch, k = inputs.shape
    h1 = W0.shape[1]
    return pl.pallas_call(
        _mlp_kernel,
        grid=(batch // _BB,),
        in_specs=[
            pl.BlockSpec((2, _BB, k), lambda i: (0, i, 0)),
            pl.BlockSpec((k, h1), lambda i: (0, 0)),
            pl.BlockSpec((k, h1), lambda i: (0, 0)),
            pl.BlockSpec((1, h1), lambda i: (0, 0)),
            pl.BlockSpec((1, h1), lambda i: (0, 0)),
            pl.BlockSpec((1, 1), lambda i: (0, 0)),
        ],
        out_specs=pl.BlockSpec((_BB, 1), lambda i: (i, 0)),
        out_shape=jax.ShapeDtypeStruct((batch, 1), jnp.float32),
        compiler_params=pltpu.CompilerParams(
            dimension_semantics=("parallel",),
        ),
    )(inputs, W0[:k], W0[k:], b0.reshape(1, h1), W1.reshape(1, h1),
      b1.reshape(1, 1))


# speedup vs baseline: 1.3467x; 1.3467x over previous
import jax
import jax.numpy as jnp
from jax.experimental import pallas as pl
from jax.experimental.pallas import tpu as pltpu

_BB = 2048


def _mlp_kernel(x_ref, w0a_ref, w0b_ref, b0_ref, w1_ref, b1_ref, out_ref):
    h = jnp.dot(x_ref[0], w0a_ref[...], preferred_element_type=jnp.float32)
    h += jnp.dot(x_ref[1], w0b_ref[...], preferred_element_type=jnp.float32)
    h = jnp.maximum(h + b0_ref[...], 0.0)
    r = jnp.sum(h * w1_ref[...], axis=1) + b1_ref[0, 0]
    out_ref[...] = r.reshape(out_ref.shape)

def kernel(inputs, W0, b0, W1, b1):
    _, batch, k = inputs.shape
    h1 = W0.shape[1]
    out = pl.pallas_call(
        _mlp_kernel,
        grid=(batch // _BB,),
        in_specs=[
            pl.BlockSpec((2, _BB, k), lambda i: (0, i, 0)),
            pl.BlockSpec((k, h1), lambda i: (0, 0)),
            pl.BlockSpec((k, h1), lambda i: (0, 0)),
            pl.BlockSpec((1, h1), lambda i: (0, 0)),
            pl.BlockSpec((1, h1), lambda i: (0, 0)),
            pl.BlockSpec((1, 1), lambda i: (0, 0)),
        ],
        out_specs=pl.BlockSpec((_BB // 128, 128), lambda i: (i, 0)),
        out_shape=jax.ShapeDtypeStruct((batch // 128, 128), jnp.float32),
        compiler_params=pltpu.CompilerParams(
            dimension_semantics=("parallel",),
        ),
    )(inputs, W0[:k], W0[k:], b0.reshape(1, h1), W1.reshape(1, h1),
      b1.reshape(1, 1))
    return out.reshape(batch, 1)


# speedup vs baseline: 1.5071x; 1.1191x over previous
import jax
import jax.numpy as jnp
from jax.experimental import pallas as pl
from jax.experimental.pallas import tpu as pltpu

_BB = 8192


def _mlp_kernel(x_ref, w0a_ref, w0b_ref, b0_ref, w1_ref, b1_ref, out_ref):
    h = jnp.dot(x_ref[0], w0a_ref[...], preferred_element_type=jnp.float32)
    h += jnp.dot(x_ref[1], w0b_ref[...], preferred_element_type=jnp.float32)
    h = jnp.maximum(h + b0_ref[...], 0.0)
    r = jnp.sum(h * w1_ref[...], axis=1) + b1_ref[0, 0]
    out_ref[...] = r.reshape(out_ref.shape)

def kernel(inputs, W0, b0, W1, b1):
    _, batch, k = inputs.shape
    h1 = W0.shape[1]
    out = pl.pallas_call(
        _mlp_kernel,
        grid=(batch // _BB,),
        in_specs=[
            pl.BlockSpec((2, _BB, k), lambda i: (0, i, 0)),
            pl.BlockSpec((k, h1), lambda i: (0, 0)),
            pl.BlockSpec((k, h1), lambda i: (0, 0)),
            pl.BlockSpec((1, h1), lambda i: (0, 0)),
            pl.BlockSpec((1, h1), lambda i: (0, 0)),
            pl.BlockSpec((1, 1), lambda i: (0, 0)),
        ],
        out_specs=pl.BlockSpec((_BB // 128, 128), lambda i: (i, 0)),
        out_shape=jax.ShapeDtypeStruct((batch // 128, 128), jnp.float32),
        compiler_params=pltpu.CompilerParams(
            dimension_semantics=("parallel",),
        ),
    )(inputs, W0[:k], W0[k:], b0.reshape(1, h1), W1.reshape(1, h1),
      b1.reshape(1, 1))
    return out.reshape(batch, 1)
